# Initial kernel scaffold; baseline (speedup 1.0000x reference)
import jax
import jax.numpy as jnp
from jax.experimental import pallas as pl


def kernel(x, edge_index, W1, b1, W2, b2):
    raise NotImplementedError("write your pallas kernel here")


# speedup vs baseline: 17.3493x; 17.3493x over previous
import functools

import jax
import jax.numpy as jnp
from jax import lax
from jax.experimental import pallas as pl
from jax.experimental.pallas import tpu as pltpu
from jax.experimental.pallas import tpu_sc as plsc

N = 10000
NP = 10240
DIN = 165
DR = 192
DH = 96
HID = 360
E = 320000
K = 128
CPW = 79
EP = 32 * CPW * K
CPT = EP // K // 16
NW = 32
ROWS_PT = NP // 16
ROWS_PW = NP // NW

_mesh = plsc.VectorSubcoreMesh(
    core_axis_name="c", subcore_axis_name="s", num_cores=2, num_subcores=16
)
_sc_params = pltpu.CompilerParams(use_tc_tiling_on_sc=False)


def _zero_1d(ref, n):
    @pl.loop(0, n // 16)
    def _(i):
        ref[pl.ds(i * 16, 16)] = jnp.zeros((16,), jnp.float32)


@functools.partial(
    pl.kernel,
    out_type=jax.ShapeDtypeStruct((2 * NP,), jnp.float32),
    mesh=_mesh,
    compiler_params=_sc_params,
    scratch_types=[
        pltpu.VMEM((CPW, K), jnp.int32),
        pltpu.VMEM((K,), jnp.float32),
        pltpu.VMEM((ROWS_PT,), jnp.float32),
        pltpu.VMEM_SHARED((NP,), jnp.float32),
    ],
)
def _deg_kernel(dst_hbm, deg_out, idx_v, ones_v, z_v, acc_sh):
    cid = lax.axis_index("c")
    sid = lax.axis_index("s")
    wid = cid * 16 + sid

    @pl.loop(0, K // 16)
    def _(i):
        ones_v[pl.ds(i * 16, 16)] = jnp.ones((16,), jnp.float32)

    _zero_1d(z_v, ROWS_PT)
    pltpu.sync_copy(z_v, acc_sh.at[pl.ds(sid * ROWS_PT, ROWS_PT)])
    pltpu.sync_copy(dst_hbm.at[wid], idx_v)
    plsc.subcore_barrier()

    @pl.loop(0, CPW)
    def _(c):
        pltpu.sync_copy(ones_v, acc_sh.at[idx_v.at[c]], add=True)

    plsc.subcore_barrier()
    pltpu.sync_copy(
        acc_sh.at[pl.ds(sid * ROWS_PT, ROWS_PT)],
        deg_out.at[pl.ds(cid * NP + sid * ROWS_PT, ROWS_PT)],
    )


def _prep_body(deg_ref, x_ref, dinv_ref, y_ref):
    d = deg_ref[...]
    di = lax.rsqrt(d[0] + d[1] + 1.0)
    dinv_ref[...] = di
    y_ref[...] = x_ref[...] * di


def _prep(deg2, xr):
    bm = 512
    return pl.pallas_call(
        _prep_body,
        grid=(NP // bm,),
        in_specs=[
            pl.BlockSpec((2, bm, 1), lambda i: (0, i, 0)),
            pl.BlockSpec((bm, DR), lambda i: (i, 0)),
        ],
        out_specs=[
            pl.BlockSpec((bm, 1), lambda i: (i, 0)),
            pl.BlockSpec((bm, DR), lambda i: (i, 0)),
        ],
        out_shape=[
            jax.ShapeDtypeStruct((NP, 1), jnp.float32),
            jax.ShapeDtypeStruct((NP, DR), jnp.float32),
        ],
    )(deg2, xr)


@functools.partial(
    pl.kernel,
    out_type=jax.ShapeDtypeStruct((2, NP, DH), jnp.float32),
    mesh=_mesh,
    compiler_params=_sc_params,
    scratch_types=[
        pltpu.VMEM((CPT, K), jnp.int32),
        pltpu.VMEM((CPT, K), jnp.int32),
        pltpu.VMEM((K, DH), jnp.float32),
        pltpu.VMEM_SHARED((NP, DH), jnp.float32),
    ],
)
def _s1_kernel(srcA_hbm, srcB_hbm, dst_hbm, y_hbm, acc_out, si_v, di_v,
               rows_v, acc_sh):
    cid = lax.axis_index("c")
    sid = lax.axis_index("s")

    @pl.loop(0, K)
    def _(r):
        @pl.loop(0, DH // 16)
        def _(j):
            rows_v[r, pl.ds(j * 16, 16)] = jnp.zeros((16,), jnp.float32)

    @pl.loop(0, ROWS_PT // K)
    def _(b):
        pltpu.sync_copy(rows_v, acc_sh.at[pl.ds(sid * ROWS_PT + b * K, K)])

    @pl.when(cid == 0)
    def _():
        pltpu.sync_copy(srcA_hbm.at[sid], si_v)

    @pl.when(cid == 1)
    def _():
        pltpu.sync_copy(srcB_hbm.at[sid], si_v)

    pltpu.sync_copy(dst_hbm.at[sid], di_v)
    plsc.subcore_barrier()

    @pl.loop(0, CPT)
    def _(c):
        pltpu.sync_copy(y_hbm.at[si_v.at[c]], rows_v)
        pltpu.sync_copy(rows_v, acc_sh.at[di_v.at[c]], add=True)

    plsc.subcore_barrier()
    pltpu.sync_copy(
        acc_sh.at[pl.ds(sid * ROWS_PT, ROWS_PT)],
        acc_out.at[cid, pl.ds(sid * ROWS_PT, ROWS_PT)],
    )


def _mm_body(acc_ref, y_ref, dinv_ref, w1_ref, b1_ref, w2_ref, u_ref):
    di = dinv_ref[...]
    a2 = acc_ref[...]
    a = (jnp.concatenate([a2[0], a2[1]], axis=1) + y_ref[...]) * di
    h = jnp.dot(a, w1_ref[...], preferred_element_type=jnp.float32,
                precision=lax.Precision.HIGHEST)
    h = jnp.maximum(h + b1_ref[...], 0.0)
    t = jnp.dot(h, w2_ref[...], preferred_element_type=jnp.float32,
                precision=lax.Precision.HIGHEST)
    u_ref[...] = t * di


def _mm(acc2, y, dinv, w1r, b1r, w2):
    bm = 512
    return pl.pallas_call(
        _mm_body,
        grid=(NP // bm,),
        in_specs=[
            pl.BlockSpec((2, bm, DH), lambda i: (0, i, 0)),
            pl.BlockSpec((bm, DR), lambda i: (i, 0)),
            pl.BlockSpec((bm, 1), lambda i: (i, 0)),
            pl.BlockSpec((DR, HID), lambda i: (0, 0)),
            pl.BlockSpec((1, HID), lambda i: (0, 0)),
            pl.BlockSpec((HID, 1), lambda i: (0, 0)),
        ],
        out_specs=pl.BlockSpec((bm, 1), lambda i: (i, 0)),
        out_shape=jax.ShapeDtypeStruct((NP, 1), jnp.float32),
    )(acc2, y, dinv, w1r, b1r, w2)


@functools.partial(
    pl.kernel,
    out_type=jax.ShapeDtypeStruct((NP,), jnp.float32),
    mesh=_mesh,
    compiler_params=_sc_params,
    scratch_types=[
        pltpu.VMEM((CPT, K), jnp.int32),
        pltpu.VMEM((CPT, K), jnp.int32),
        pltpu.VMEM((K,), jnp.float32),
        pltpu.VMEM((ROWS_PT,), jnp.float32),
        pltpu.VMEM((ROWS_PW,), jnp.float32),
        pltpu.VMEM((ROWS_PW,), jnp.float32),
        pltpu.VMEM((ROWS_PW,), jnp.float32),
        pltpu.VMEM((ROWS_PW,), jnp.float32),
        pltpu.VMEM((16,), jnp.float32),
        pltpu.VMEM_SHARED((NP,), jnp.float32),
    ],
)
def _s2_kernel(src_hbm, dst_hbm, u_hbm, dinv_hbm, b2_hbm, o_hbm,
               si_v, di_v, uv_v, z_v, s2b_v, ub_v, db_v, ob_v, b2_v, acc_sh):
    cid = lax.axis_index("c")
    sid = lax.axis_index("s")
    wid = cid * 16 + sid

    _zero_1d(z_v, ROWS_PT)
    pltpu.sync_copy(z_v, acc_sh.at[pl.ds(sid * ROWS_PT, ROWS_PT)])
    pltpu.sync_copy(src_hbm.at[sid], si_v)
    pltpu.sync_copy(dst_hbm.at[sid], di_v)
    pltpu.sync_copy(b2_hbm, b2_v)
    plsc.subcore_barrier()

    @pl.loop(0, CPT)
    def _(c):
        pltpu.sync_copy(u_hbm.at[si_v.at[c]], uv_v)
        pltpu.sync_copy(uv_v, acc_sh.at[di_v.at[c]], add=True)

    plsc.subcore_barrier()

    base = wid * ROWS_PW
    pltpu.sync_copy(acc_sh.at[pl.ds(base, ROWS_PW)], s2b_v)
    pltpu.sync_copy(u_hbm.at[pl.ds(base, ROWS_PW)], ub_v)
    pltpu.sync_copy(dinv_hbm.at[pl.ds(base, ROWS_PW)], db_v)

    @pl.loop(0, ROWS_PW // 16)
    def _(k):
        s2 = s2b_v[pl.ds(k * 16, 16)]
        uu = ub_v[pl.ds(k * 16, 16)]
        dd = db_v[pl.ds(k * 16, 16)]
        b2 = b2_v[pl.ds(0, 16)]
        zz = dd * (s2 + uu) + b2
        ob_v[pl.ds(k * 16, 16)] = 1.0 / (1.0 + jnp.exp(-zz))

    pltpu.sync_copy(ob_v, o_hbm.at[pl.ds(base, ROWS_PW)])


def kernel(x, edge_index, W1, b1, W2, b2):
    src = edge_index[0].astype(jnp.int32)
    dst = edge_index[1].astype(jnp.int32)
    pad_e = EP - E
    fdst = N + jnp.arange(pad_e, dtype=jnp.int32) % 16
    fsrc = jnp.arange(pad_e, dtype=jnp.int32) % N
    src_p = jnp.concatenate([src, fsrc])
    dst_p = jnp.concatenate([dst, fdst])
    dst_w = dst_p.reshape(NW, CPW, K)
    src_t = src_p.reshape(16, CPT, K)
    dst_t = dst_p.reshape(16, CPT, K)
    srcA = (2 * src_p).reshape(16, CPT, K)
    srcB = (2 * src_p + 1).reshape(16, CPT, K)

    xr = jnp.zeros((NP, DR), jnp.float32)
    xr = xr.at[:N, 0:88].set(x[:, 0:88])
    xr = xr.at[:N, 96:173].set(x[:, 88:165])
    w1r = jnp.zeros((DR, HID), jnp.float32)
    w1r = w1r.at[0:88].set(W1[0:88])
    w1r = w1r.at[96:173].set(W1[88:165])
    b1r = b1.reshape(1, HID)
    b2v = jnp.broadcast_to(b2, (16,))

    deg2 = _deg_kernel(dst_w)
    dinv, y = _prep(deg2.reshape(2, NP, 1), xr)
    y2 = y.reshape(2 * NP, DH)
    acc2 = _s1_kernel(srcA, srcB, dst_t, y2)
    u = _mm(acc2, y, dinv, w1r, b1r, W2)
    o = _s2_kernel(src_t, dst_t, u.reshape(NP), dinv.reshape(NP), b2v)
    return o[:N].reshape(N, 1)


# speedup vs baseline: 20.4758x; 1.1802x over previous
import functools

import jax
import jax.numpy as jnp
from jax import lax
from jax.experimental import pallas as pl
from jax.experimental.pallas import tpu as pltpu
from jax.experimental.pallas import tpu_sc as plsc

N = 10000
NP = 10240
DIN = 165
DR = 192
DH = 96
HID = 360
E = 320000
K = 128
CPW = 84
EP = 32 * CPW * K
CPT = EP // K // 16
NW = 32
ROWS_PT = NP // 16
ROWS_PW = NP // NW

_mesh = plsc.VectorSubcoreMesh(
    core_axis_name="c", subcore_axis_name="s", num_cores=2, num_subcores=16
)
_sc_params = pltpu.CompilerParams(use_tc_tiling_on_sc=False)
_sc_params_nl = pltpu.CompilerParams(use_tc_tiling_on_sc=False,
                                     needs_layout_passes=False)


def _zero_1d(ref, n):
    @pl.loop(0, n // 16)
    def _(i):
        ref[pl.ds(i * 16, 16)] = jnp.zeros((16,), jnp.float32)


@functools.partial(
    pl.kernel,
    out_type=jax.ShapeDtypeStruct((2 * NP,), jnp.float32),
    mesh=_mesh,
    compiler_params=_sc_params,
    scratch_types=[
        pltpu.VMEM((CPW, K), jnp.int32),
        pltpu.VMEM((K,), jnp.float32),
        pltpu.VMEM((ROWS_PT,), jnp.float32),
        pltpu.VMEM_SHARED((NP,), jnp.float32),
    ],
)
def _deg_kernel(dst_hbm, deg_out, idx_v, ones_v, z_v, acc_sh):
    cid = lax.axis_index("c")
    sid = lax.axis_index("s")
    wid = cid * 16 + sid

    @pl.loop(0, K // 16)
    def _(i):
        ones_v[pl.ds(i * 16, 16)] = jnp.ones((16,), jnp.float32)

    _zero_1d(z_v, ROWS_PT)
    pltpu.sync_copy(z_v, acc_sh.at[pl.ds(sid * ROWS_PT, ROWS_PT)])
    pltpu.sync_copy(dst_hbm.at[wid], idx_v)
    plsc.subcore_barrier()

    @pl.loop(0, CPW)
    def _(c):
        pltpu.sync_copy(ones_v, acc_sh.at[idx_v.at[c]], add=True)

    plsc.subcore_barrier()
    pltpu.sync_copy(
        acc_sh.at[pl.ds(sid * ROWS_PT, ROWS_PT)],
        deg_out.at[pl.ds(cid * NP + sid * ROWS_PT, ROWS_PT)],
    )


def _prep_body(deg_ref, x_ref, dinv_ref, y_ref):
    d = deg_ref[...]
    di = lax.rsqrt(d[0] + d[1] + 1.0)
    dinv_ref[...] = di
    y_ref[...] = x_ref[...] * di


def _prep(deg2, xr):
    bm = 512
    return pl.pallas_call(
        _prep_body,
        grid=(NP // bm,),
        in_specs=[
            pl.BlockSpec((2, bm, 1), lambda i: (0, i, 0)),
            pl.BlockSpec((bm, DR), lambda i: (i, 0)),
        ],
        out_specs=[
            pl.BlockSpec((bm, 1), lambda i: (i, 0)),
            pl.BlockSpec((bm, DR), lambda i: (i, 0)),
        ],
        out_shape=[
            jax.ShapeDtypeStruct((NP, 1), jnp.float32),
            jax.ShapeDtypeStruct((NP, DR), jnp.float32),
        ],
    )(deg2, xr)


@functools.partial(
    pl.kernel,
    out_type=jax.ShapeDtypeStruct((2, NP, DH), jnp.float32),
    mesh=_mesh,
    compiler_params=_sc_params,
    scratch_types=[
        pltpu.VMEM((CPT + 2, K), jnp.int32),
        pltpu.VMEM((CPT, K), jnp.int32),
        pltpu.VMEM((K, DH), jnp.float32),
        pltpu.VMEM((K, DH), jnp.float32),
        pltpu.SemaphoreType.DMA,
        pltpu.VMEM_SHARED((NP, DH), jnp.float32),
    ],
)
def _s1_kernel(srcA_hbm, srcB_hbm, dst_hbm, y_hbm, acc_out, si_v, di_v,
               rows_v, rows1_v, gsem, acc_sh):
    cid = lax.axis_index("c")
    sid = lax.axis_index("s")

    @pl.loop(0, K)
    def _(r):
        @pl.loop(0, DH // 16)
        def _(j):
            rows_v[r, pl.ds(j * 16, 16)] = jnp.zeros((16,), jnp.float32)

    @pl.loop(0, ROWS_PT // K)
    def _(b):
        pltpu.sync_copy(rows_v, acc_sh.at[pl.ds(sid * ROWS_PT + b * K, K)])

    @pl.when(cid == 0)
    def _():
        pltpu.sync_copy(srcA_hbm.at[sid], si_v.at[pl.ds(0, CPT)])

    @pl.when(cid == 1)
    def _():
        pltpu.sync_copy(srcB_hbm.at[sid], si_v.at[pl.ds(0, CPT)])

    @pl.loop(0, 2 * K // 16)
    def _(i):
        si_v[CPT + i // 8, pl.ds((i % 8) * 16, 16)] = jnp.zeros((16,), jnp.int32)

    pltpu.sync_copy(dst_hbm.at[sid], di_v)
    plsc.subcore_barrier()

    pltpu.async_copy(y_hbm.at[si_v.at[0]], rows_v, gsem).wait()

    @pl.loop(0, CPT, step=2)
    def _(c):
        g1 = pltpu.async_copy(y_hbm.at[si_v.at[c + 1]], rows1_v, gsem)
        pltpu.sync_copy(rows_v, acc_sh.at[di_v.at[c]], add=True)
        g1.wait()
        g2 = pltpu.async_copy(y_hbm.at[si_v.at[c + 2]], rows_v, gsem)
        pltpu.sync_copy(rows1_v, acc_sh.at[di_v.at[c + 1]], add=True)
        g2.wait()

    plsc.subcore_barrier()
    pltpu.sync_copy(
        acc_sh.at[pl.ds(sid * ROWS_PT, ROWS_PT)],
        acc_out.at[cid, pl.ds(sid * ROWS_PT, ROWS_PT)],
    )


def _mm_body(acc_ref, y_ref, dinv_ref, w1_ref, b1_ref, w2_ref, u_ref):
    di = dinv_ref[...]
    a2 = acc_ref[...]
    a = (jnp.concatenate([a2[0], a2[1]], axis=1) + y_ref[...]) * di
    h = jnp.dot(a, w1_ref[...], preferred_element_type=jnp.float32,
                precision=lax.Precision.HIGHEST)
    h = jnp.maximum(h + b1_ref[...], 0.0)
    t = jnp.dot(h, w2_ref[...], preferred_element_type=jnp.float32,
                precision=lax.Precision.HIGHEST)
    u_ref[...] = t * di


def _mm(acc2, y, dinv, w1r, b1r, w2):
    bm = 512
    return pl.pallas_call(
        _mm_body,
        grid=(NP // bm,),
        in_specs=[
            pl.BlockSpec((2, bm, DH), lambda i: (0, i, 0)),
            pl.BlockSpec((bm, DR), lambda i: (i, 0)),
            pl.BlockSpec((bm, 1), lambda i: (i, 0)),
            pl.BlockSpec((DR, HID), lambda i: (0, 0)),
            pl.BlockSpec((1, HID), lambda i: (0, 0)),
            pl.BlockSpec((HID, 1), lambda i: (0, 0)),
        ],
        out_specs=pl.BlockSpec((bm, 1), lambda i: (i, 0)),
        out_shape=jax.ShapeDtypeStruct((NP, 1), jnp.float32),
    )(acc2, y, dinv, w1r, b1r, w2)


@functools.partial(
    pl.kernel,
    out_type=jax.ShapeDtypeStruct((NP,), jnp.float32),
    mesh=_mesh,
    compiler_params=_sc_params_nl,
    scratch_types=[
        pltpu.VMEM((CPT, K), jnp.int32),
        pltpu.VMEM((CPT, K), jnp.int32),
        pltpu.VMEM((CPT, K), jnp.float32),
        pltpu.VMEM((NP,), jnp.float32),
        pltpu.VMEM((ROWS_PT,), jnp.float32),
        pltpu.VMEM((ROWS_PW,), jnp.float32),
        pltpu.VMEM((ROWS_PW,), jnp.float32),
        pltpu.VMEM((ROWS_PW,), jnp.float32),
        pltpu.VMEM((ROWS_PW,), jnp.float32),
        pltpu.VMEM((16,), jnp.float32),
        pltpu.SemaphoreType.DMA,
        pltpu.VMEM_SHARED((NP,), jnp.float32),
    ],
)
def _s2_kernel(src_hbm, dst_hbm, u_hbm, dinv_hbm, b2_hbm, o_hbm,
               si_v, di_v, upd_v, u_v, z_v, s2b_v, ub_v, db_v, ob_v, b2_v,
               ssem, acc_sh):
    cid = lax.axis_index("c")
    sid = lax.axis_index("s")
    wid = cid * 16 + sid

    _zero_1d(z_v, ROWS_PT)
    pltpu.sync_copy(z_v, acc_sh.at[pl.ds(sid * ROWS_PT, ROWS_PT)])
    pltpu.sync_copy(src_hbm.at[sid], si_v)
    pltpu.sync_copy(dst_hbm.at[sid], di_v)
    pltpu.sync_copy(b2_hbm, b2_v)
    pltpu.sync_copy(u_hbm, u_v)

    @pl.loop(0, CPT)
    def _(c):
        @pl.loop(0, K // 16)
        def _(j):
            sidx = si_v[c, pl.ds(j * 16, 16)]
            upd_v[c, pl.ds(j * 16, 16)] = plsc.load_gather(u_v, [sidx])

    plsc.subcore_barrier()

    @pl.loop(0, CPT, step=8)
    def _(c0):
        descs = [
            pltpu.async_copy(upd_v.at[c0 + b], acc_sh.at[di_v.at[c0 + b]],
                             ssem, add=True)
            for b in range(8)
        ]
        for d in descs:
            d.wait()

    plsc.subcore_barrier()

    base = wid * ROWS_PW
    pltpu.sync_copy(acc_sh.at[pl.ds(base, ROWS_PW)], s2b_v)
    pltpu.sync_copy(u_hbm.at[pl.ds(base, ROWS_PW)], ub_v)
    pltpu.sync_copy(dinv_hbm.at[pl.ds(base, ROWS_PW)], db_v)

    @pl.loop(0, ROWS_PW // 16)
    def _(k):
        s2 = s2b_v[pl.ds(k * 16, 16)]
        uu = ub_v[pl.ds(k * 16, 16)]
        dd = db_v[pl.ds(k * 16, 16)]
        b2 = b2_v[pl.ds(0, 16)]
        zz = dd * (s2 + uu) + b2
        ob_v[pl.ds(k * 16, 16)] = 1.0 / (1.0 + jnp.exp(-zz))

    pltpu.sync_copy(ob_v, o_hbm.at[pl.ds(base, ROWS_PW)])


def kernel(x, edge_index, W1, b1, W2, b2):
    src = edge_index[0].astype(jnp.int32)
    dst = edge_index[1].astype(jnp.int32)
    pad_e = EP - E
    fdst = 10016 + jnp.arange(pad_e, dtype=jnp.int32) % 224
    fsrc = jnp.arange(pad_e, dtype=jnp.int32) % N
    src_p = jnp.concatenate([src, fsrc])
    dst_p = jnp.concatenate([dst, fdst])
    dst_w = dst_p.reshape(NW, CPW, K)
    src_t = src_p.reshape(16, CPT, K)
    dst_t = dst_p.reshape(16, CPT, K)
    srcA = (2 * src_p).reshape(16, CPT, K)
    srcB = (2 * src_p + 1).reshape(16, CPT, K)

    xr = jnp.zeros((NP, DR), jnp.float32)
    xr = xr.at[:N, 0:88].set(x[:, 0:88])
    xr = xr.at[:N, 96:173].set(x[:, 88:165])
    w1r = jnp.zeros((DR, HID), jnp.float32)
    w1r = w1r.at[0:88].set(W1[0:88])
    w1r = w1r.at[96:173].set(W1[88:165])
    b1r = b1.reshape(1, HID)
    b2v = jnp.broadcast_to(b2, (16,))

    deg2 = _deg_kernel(dst_w)
    dinv, y = _prep(deg2.reshape(2, NP, 1), xr)
    y2 = y.reshape(2 * NP, DH)
    acc2 = _s1_kernel(srcA, srcB, dst_t, y2)
    u = _mm(acc2, y, dinv, w1r, b1r, W2)
    o = _s2_kernel(src_t, dst_t, u.reshape(NP), dinv.reshape(NP), b2v)
    return o[:N].reshape(N, 1)


# speedup vs baseline: 28.9280x; 1.4128x over previous
import functools

import jax
import jax.numpy as jnp
from jax import lax
from jax.experimental import pallas as pl
from jax.experimental.pallas import tpu as pltpu
from jax.experimental.pallas import tpu_sc as plsc

N = 10000
NP = 10240
DIN = 165
DR = 192
DH = 96
HID = 360
E = 320000
K = 128
CPW = 84
EP = 32 * CPW * K
CPT = EP // K // 16
KS = 64
CPS = EP // KS // 16
NW = 32
ROWS_PT = NP // 16
ROWS_PW = NP // NW

_mesh = plsc.VectorSubcoreMesh(
    core_axis_name="c", subcore_axis_name="s", num_cores=2, num_subcores=16
)
_sc_params = pltpu.CompilerParams(use_tc_tiling_on_sc=False)
_sc_params_nl = pltpu.CompilerParams(use_tc_tiling_on_sc=False,
                                     needs_layout_passes=False)


def _zero_1d(ref, n):
    @pl.loop(0, n // 16)
    def _(i):
        ref[pl.ds(i * 16, 16)] = jnp.zeros((16,), jnp.float32)


@functools.partial(
    pl.kernel,
    out_type=jax.ShapeDtypeStruct((2 * NP,), jnp.float32),
    mesh=_mesh,
    compiler_params=_sc_params,
    scratch_types=[
        pltpu.VMEM((CPW, K), jnp.int32),
        pltpu.VMEM((K,), jnp.float32),
        pltpu.VMEM((ROWS_PT,), jnp.float32),
        pltpu.VMEM_SHARED((NP,), jnp.float32),
    ],
)
def _deg_kernel(dst_hbm, deg_out, idx_v, ones_v, z_v, acc_sh):
    cid = lax.axis_index("c")
    sid = lax.axis_index("s")
    wid = cid * 16 + sid

    @pl.loop(0, K // 16)
    def _(i):
        ones_v[pl.ds(i * 16, 16)] = jnp.ones((16,), jnp.float32)

    _zero_1d(z_v, ROWS_PT)
    pltpu.sync_copy(z_v, acc_sh.at[pl.ds(sid * ROWS_PT, ROWS_PT)])
    pltpu.sync_copy(dst_hbm.at[wid], idx_v)
    plsc.subcore_barrier()

    @pl.loop(0, CPW)
    def _(c):
        pltpu.sync_copy(ones_v, acc_sh.at[idx_v.at[c]], add=True)

    plsc.subcore_barrier()
    pltpu.sync_copy(
        acc_sh.at[pl.ds(sid * ROWS_PT, ROWS_PT)],
        deg_out.at[pl.ds(cid * NP + sid * ROWS_PT, ROWS_PT)],
    )


def _prep_body(deg_ref, x_ref, dinv_ref, y_ref):
    d = deg_ref[...]
    di = lax.rsqrt(d[0] + d[1] + 1.0)
    dinv_ref[...] = di
    y_ref[...] = x_ref[...] * di


def _prep(deg2, xr):
    bm = 512
    return pl.pallas_call(
        _prep_body,
        grid=(NP // bm,),
        in_specs=[
            pl.BlockSpec((2, bm, 1), lambda i: (0, i, 0)),
            pl.BlockSpec((bm, DR), lambda i: (i, 0)),
        ],
        out_specs=[
            pl.BlockSpec((bm, 1), lambda i: (i, 0)),
            pl.BlockSpec((bm, DR), lambda i: (i, 0)),
        ],
        out_shape=[
            jax.ShapeDtypeStruct((NP, 1), jnp.float32),
            jax.ShapeDtypeStruct((NP, DR), jnp.float32),
        ],
    )(deg2, xr)


@functools.partial(
    pl.kernel,
    out_type=jax.ShapeDtypeStruct((2, NP, DH), jnp.float32),
    mesh=_mesh,
    compiler_params=_sc_params,
    scratch_types=[
        pltpu.VMEM((CPS, KS), jnp.int32),
        pltpu.VMEM((CPS, KS), jnp.int32),
        [pltpu.VMEM((KS, DH), jnp.float32)] * 4,
        [pltpu.SemaphoreType.DMA] * 4,
        [pltpu.SemaphoreType.DMA] * 4,
        pltpu.VMEM_SHARED((NP, DH), jnp.float32),
    ],
)
def _s1_kernel(srcA_hbm, srcB_hbm, dst_hbm, y_hbm, acc_out, si_v, di_v,
               rows, gsem, ssem, acc_sh):
    cid = lax.axis_index("c")
    sid = lax.axis_index("s")

    @pl.loop(0, KS)
    def _(r):
        @pl.loop(0, DH // 16)
        def _(j):
            rows[0][r, pl.ds(j * 16, 16)] = jnp.zeros((16,), jnp.float32)

    @pl.loop(0, ROWS_PT // KS)
    def _(b):
        pltpu.sync_copy(rows[0], acc_sh.at[pl.ds(sid * ROWS_PT + b * KS, KS)])

    @pl.when(cid == 0)
    def _():
        pltpu.sync_copy(srcA_hbm.at[sid], si_v)

    @pl.when(cid == 1)
    def _():
        pltpu.sync_copy(srcB_hbm.at[sid], si_v)

    pltpu.sync_copy(dst_hbm.at[sid], di_v)
    plsc.subcore_barrier()

    def fire_gather(c, b):
        pltpu.async_copy(y_hbm.at[si_v.at[c]], rows[b], gsem[b])

    def drain_gather(b):
        pltpu.make_async_copy(y_hbm.at[si_v.at[0]], rows[b], gsem[b]).wait()

    def fire_scatter(c, b):
        pltpu.async_copy(rows[b], acc_sh.at[di_v.at[c]], ssem[b], add=True)

    def drain_scatter(b):
        pltpu.make_async_copy(rows[b], acc_sh.at[di_v.at[0]], ssem[b]).wait()

    fire_gather(0, 0)
    fire_gather(1, 1)

    @pl.loop(2, CPS + 2, step=4)
    def _(c0):
        for o in range(4):
            c = c0 + o
            bg = (2 + o) % 4
            bs = o % 4

            @pl.when(c >= 4)
            def _():
                drain_scatter(bg)

            @pl.when(c < CPS)
            def _():
                fire_gather(c, bg)

            drain_gather(bs)
            fire_scatter(c - 2, bs)

    drain_scatter(2)
    drain_scatter(3)

    plsc.subcore_barrier()
    pltpu.sync_copy(
        acc_sh.at[pl.ds(sid * ROWS_PT, ROWS_PT)],
        acc_out.at[cid, pl.ds(sid * ROWS_PT, ROWS_PT)],
    )


def _mm_body(acc_ref, y_ref, dinv_ref, w1_ref, b1_ref, w2_ref, u_ref):
    di = dinv_ref[...]
    a2 = acc_ref[...]
    a = (jnp.concatenate([a2[0], a2[1]], axis=1) + y_ref[...]) * di
    h = jnp.dot(a, w1_ref[...], preferred_element_type=jnp.float32,
                precision=lax.Precision.HIGHEST)
    h = jnp.maximum(h + b1_ref[...], 0.0)
    t = jnp.dot(h, w2_ref[...], preferred_element_type=jnp.float32,
                precision=lax.Precision.HIGHEST)
    u_ref[...] = t * di


def _mm(acc2, y, dinv, w1r, b1r, w2):
    bm = 512
    return pl.pallas_call(
        _mm_body,
        grid=(NP // bm,),
        in_specs=[
            pl.BlockSpec((2, bm, DH), lambda i: (0, i, 0)),
            pl.BlockSpec((bm, DR), lambda i: (i, 0)),
            pl.BlockSpec((bm, 1), lambda i: (i, 0)),
            pl.BlockSpec((DR, HID), lambda i: (0, 0)),
            pl.BlockSpec((1, HID), lambda i: (0, 0)),
            pl.BlockSpec((HID, 1), lambda i: (0, 0)),
        ],
        out_specs=pl.BlockSpec((bm, 1), lambda i: (i, 0)),
        out_shape=jax.ShapeDtypeStruct((NP, 1), jnp.float32),
    )(acc2, y, dinv, w1r, b1r, w2)


@functools.partial(
    pl.kernel,
    out_type=jax.ShapeDtypeStruct((NP,), jnp.float32),
    mesh=_mesh,
    compiler_params=_sc_params_nl,
    scratch_types=[
        pltpu.VMEM((CPT, K), jnp.int32),
        pltpu.VMEM((CPT, K), jnp.int32),
        pltpu.VMEM((CPT, K), jnp.float32),
        pltpu.VMEM((NP,), jnp.float32),
        pltpu.VMEM((ROWS_PT,), jnp.float32),
        pltpu.VMEM((ROWS_PW,), jnp.float32),
        pltpu.VMEM((ROWS_PW,), jnp.float32),
        pltpu.VMEM((ROWS_PW,), jnp.float32),
        pltpu.VMEM((ROWS_PW,), jnp.float32),
        pltpu.VMEM((16,), jnp.float32),
        pltpu.SemaphoreType.DMA,
        pltpu.VMEM_SHARED((NP,), jnp.float32),
    ],
)
def _s2_kernel(src_hbm, dst_hbm, u_hbm, dinv_hbm, b2_hbm, o_hbm,
               si_v, di_v, upd_v, u_v, z_v, s2b_v, ub_v, db_v, ob_v, b2_v,
               ssem, acc_sh):
    cid = lax.axis_index("c")
    sid = lax.axis_index("s")
    wid = cid * 16 + sid

    _zero_1d(z_v, ROWS_PT)
    pltpu.sync_copy(z_v, acc_sh.at[pl.ds(sid * ROWS_PT, ROWS_PT)])
    pltpu.sync_copy(src_hbm.at[sid], si_v)
    pltpu.sync_copy(dst_hbm.at[sid], di_v)
    pltpu.sync_copy(b2_hbm, b2_v)
    pltpu.sync_copy(u_hbm, u_v)

    @pl.loop(0, CPT)
    def _(c):
        @pl.loop(0, K // 16)
        def _(j):
            sidx = si_v[c, pl.ds(j * 16, 16)]
            upd_v[c, pl.ds(j * 16, 16)] = plsc.load_gather(u_v, [sidx])

    plsc.subcore_barrier()

    @pl.loop(0, CPT, step=8)
    def _(c0):
        descs = [
            pltpu.async_copy(upd_v.at[c0 + b], acc_sh.at[di_v.at[c0 + b]],
                             ssem, add=True)
            for b in range(8)
        ]
        for d in descs:
            d.wait()

    plsc.subcore_barrier()

    base = wid * ROWS_PW
    pltpu.sync_copy(acc_sh.at[pl.ds(base, ROWS_PW)], s2b_v)
    pltpu.sync_copy(u_hbm.at[pl.ds(base, ROWS_PW)], ub_v)
    pltpu.sync_copy(dinv_hbm.at[pl.ds(base, ROWS_PW)], db_v)

    @pl.loop(0, ROWS_PW // 16)
    def _(k):
        s2 = s2b_v[pl.ds(k * 16, 16)]
        uu = ub_v[pl.ds(k * 16, 16)]
        dd = db_v[pl.ds(k * 16, 16)]
        b2 = b2_v[pl.ds(0, 16)]
        zz = dd * (s2 + uu) + b2
        ob_v[pl.ds(k * 16, 16)] = 1.0 / (1.0 + jnp.exp(-zz))

    pltpu.sync_copy(ob_v, o_hbm.at[pl.ds(base, ROWS_PW)])


def kernel(x, edge_index, W1, b1, W2, b2):
    src = edge_index[0].astype(jnp.int32)
    dst = edge_index[1].astype(jnp.int32)
    pad_e = EP - E
    fdst = 10016 + jnp.arange(pad_e, dtype=jnp.int32) % 224
    fsrc = jnp.arange(pad_e, dtype=jnp.int32) % N
    src_p = jnp.concatenate([src, fsrc])
    dst_p = jnp.concatenate([dst, fdst])
    dst_w = dst_p.reshape(NW, CPW, K)
    src_t = src_p.reshape(16, CPT, K)
    dst_t = dst_p.reshape(16, CPS, KS)
    srcA = (2 * src_p).reshape(16, CPS, KS)
    srcB = (2 * src_p + 1).reshape(16, CPS, KS)

    xr = jnp.zeros((NP, DR), jnp.float32)
    xr = xr.at[:N, 0:88].set(x[:, 0:88])
    xr = xr.at[:N, 96:173].set(x[:, 88:165])
    w1r = jnp.zeros((DR, HID), jnp.float32)
    w1r = w1r.at[0:88].set(W1[0:88])
    w1r = w1r.at[96:173].set(W1[88:165])
    b1r = b1.reshape(1, HID)
    b2v = jnp.broadcast_to(b2, (16,))

    deg2 = _deg_kernel(dst_w)
    dinv, y = _prep(deg2.reshape(2, NP, 1), xr)
    y2 = y.reshape(2 * NP, DH)
    acc2 = _s1_kernel(srcA, srcB, dst_t, y2)
    u = _mm(acc2, y, dinv, w1r, b1r, W2)
    dst_t2 = dst_p.reshape(16, CPT, K)
    o = _s2_kernel(src_t, dst_t2, u.reshape(NP), dinv.reshape(NP), b2v)
    return o[:N].reshape(N, 1)


# speedup vs baseline: 34.6341x; 1.1973x over previous
import functools

import jax
import jax.numpy as jnp
from jax import lax
from jax.experimental import pallas as pl
from jax.experimental.pallas import tpu as pltpu
from jax.experimental.pallas import tpu_sc as plsc

N = 10000
NP = 10240
DIN = 165
DR = 192
DH = 96
HID = 360
E = 320000
K = 128
CPW = 84
EP = 32 * CPW * K
CPT = EP // K // 16
KS = 64
CPS = EP // KS // 16
NW = 32
ROWS_PT = NP // 16
ROWS_PW = NP // NW

_mesh = plsc.VectorSubcoreMesh(
    core_axis_name="c", subcore_axis_name="s", num_cores=2, num_subcores=16
)
_sc_params = pltpu.CompilerParams(use_tc_tiling_on_sc=False)
_sc_params_nl = pltpu.CompilerParams(use_tc_tiling_on_sc=False,
                                     needs_layout_passes=False)


def _zero_1d(ref, n):
    @pl.loop(0, n // 16)
    def _(i):
        ref[pl.ds(i * 16, 16)] = jnp.zeros((16,), jnp.float32)


@functools.partial(
    pl.kernel,
    out_type=jax.ShapeDtypeStruct((NP,), jnp.float32),
    mesh=_mesh,
    compiler_params=_sc_params_nl,
    scratch_types=[
        pltpu.VMEM((CPT, K), jnp.int32),
        pltpu.VMEM((K,), jnp.float32),
        pltpu.VMEM((ROWS_PT,), jnp.float32),
        pltpu.VMEM((ROWS_PW,), jnp.float32),
        pltpu.SemaphoreType.DMA,
        pltpu.VMEM_SHARED((NP,), jnp.float32),
    ],
)
def _deg_kernel(dst_hbm, dinv_out, idx_v, ones_v, z_v, dinvb_v, hsem, acc_sh):
    cid = lax.axis_index("c")
    sid = lax.axis_index("s")
    wid = cid * 16 + sid

    @pl.loop(0, K // 16)
    def _(i):
        ones_v[pl.ds(i * 16, 16)] = jnp.ones((16,), jnp.float32)

    _zero_1d(z_v, ROWS_PT)
    pltpu.sync_copy(z_v, acc_sh.at[pl.ds(sid * ROWS_PT, ROWS_PT)])
    pltpu.sync_copy(dst_hbm.at[sid], idx_v)
    plsc.subcore_barrier()

    def fire(c):
        pltpu.async_copy(ones_v, acc_sh.at[idx_v.at[c]], hsem, add=True)

    def drain():
        pltpu.make_async_copy(ones_v, acc_sh.at[idx_v.at[0]], hsem).wait()

    fire(0)
    fire(1)

    @pl.loop(2, CPT)
    def _(c):
        drain()
        fire(c)

    drain()
    drain()
    plsc.subcore_barrier()

    pltpu.sync_copy(acc_sh.at[pl.ds(wid * ROWS_PW, ROWS_PW)], z_v.at[pl.ds(0, ROWS_PW)])

    @pl.loop(0, ROWS_PW // 16)
    def _(k):
        d = z_v[pl.ds(k * 16, 16)] + 1.0
        i = plsc.bitcast(d, jnp.int32)
        i = jnp.full((16,), 0x5F3759DF, jnp.int32) - lax.shift_right_logical(i, 1)
        r = plsc.bitcast(i, jnp.float32)
        r = r * (1.5 - 0.5 * d * r * r)
        r = r * (1.5 - 0.5 * d * r * r)
        r = r * (1.5 - 0.5 * d * r * r)
        dinvb_v[pl.ds(k * 16, 16)] = r

    pltpu.sync_copy(dinvb_v, dinv_out.at[pl.ds(wid * ROWS_PW, ROWS_PW)])


@functools.partial(
    pl.kernel,
    out_type=jax.ShapeDtypeStruct((2 * NP, DH), jnp.float32),
    mesh=_mesh,
    compiler_params=_sc_params_nl,
    scratch_types=[
        pltpu.VMEM((64, DR), jnp.float32),
        pltpu.VMEM((128, DH), jnp.float32),
        pltpu.VMEM((ROWS_PW,), jnp.float32),
        pltpu.SemaphoreType.DMA,
    ],
)
def _y2_kernel(xp_hbm, dinv_hbm, y2_out, xb_v, yb_v, dinvb_v, sem):
    cid = lax.axis_index("c")
    sid = lax.axis_index("s")
    wid = cid * 16 + sid
    base = wid * ROWS_PW
    pltpu.sync_copy(dinv_hbm.at[pl.ds(base, ROWS_PW)], dinvb_v)

    @pl.loop(0, ROWS_PW // 64)
    def _(j):
        pltpu.sync_copy(xp_hbm.at[pl.ds(base + j * 64, 64)], xb_v)

        @pl.loop(0, 64)
        def _(n):
            dv = plsc.load_gather(dinvb_v, [jnp.full((16,), j * 64 + n, jnp.int32)])
            for k in range(6):
                yb_v[2 * n, pl.ds(k * 16, 16)] = xb_v[n, pl.ds(k * 16, 16)] * dv
                yb_v[2 * n + 1, pl.ds(k * 16, 16)] = (
                    xb_v[n, pl.ds(DH + k * 16, 16)] * dv)

        pltpu.sync_copy(yb_v, y2_out.at[pl.ds(2 * base + j * 128, 128)])


@functools.partial(
    pl.kernel,
    out_type=jax.ShapeDtypeStruct((2, NP, DH), jnp.float32),
    mesh=_mesh,
    compiler_params=_sc_params,
    scratch_types=[
        pltpu.VMEM((CPS, KS), jnp.int32),
        pltpu.VMEM((CPS, KS), jnp.int32),
        [pltpu.VMEM((KS, DH), jnp.float32)] * 4,
        [pltpu.SemaphoreType.DMA] * 4,
        [pltpu.SemaphoreType.DMA] * 4,
        pltpu.VMEM_SHARED((NP, DH), jnp.float32),
    ],
)
def _s1_kernel(src_hbm, dst_hbm, y_hbm, acc_out, si_v, di_v,
               rows, gsem, ssem, acc_sh):
    cid = lax.axis_index("c")
    sid = lax.axis_index("s")

    @pl.loop(0, KS)
    def _(r):
        @pl.loop(0, DH // 16)
        def _(j):
            rows[0][r, pl.ds(j * 16, 16)] = jnp.zeros((16,), jnp.float32)

    @pl.loop(0, ROWS_PT // KS)
    def _(b):
        pltpu.sync_copy(rows[0], acc_sh.at[pl.ds(sid * ROWS_PT + b * KS, KS)])

    pltpu.sync_copy(src_hbm.at[sid], si_v)
    pltpu.sync_copy(dst_hbm.at[sid], di_v)

    @pl.loop(0, CPS)
    def _(r):
        @pl.loop(0, KS // 16)
        def _(j):
            v = si_v[r, pl.ds(j * 16, 16)]
            si_v[r, pl.ds(j * 16, 16)] = v * 2 + cid

    plsc.subcore_barrier()

    def fire_gather(c, b):
        pltpu.async_copy(y_hbm.at[si_v.at[c]], rows[b], gsem[b])

    def drain_gather(b):
        pltpu.make_async_copy(y_hbm.at[si_v.at[0]], rows[b], gsem[b]).wait()

    def fire_scatter(c, b):
        pltpu.async_copy(rows[b], acc_sh.at[di_v.at[c]], ssem[b], add=True)

    def drain_scatter(b):
        pltpu.make_async_copy(rows[b], acc_sh.at[di_v.at[0]], ssem[b]).wait()

    fire_gather(0, 0)
    fire_gather(1, 1)

    @pl.loop(2, CPS + 2, step=4)
    def _(c0):
        for o in range(4):
            c = c0 + o
            bg = (2 + o) % 4
            bs = o % 4

            @pl.when(c >= 4)
            def _():
                drain_scatter(bg)

            @pl.when(c < CPS)
            def _():
                fire_gather(c, bg)

            drain_gather(bs)
            fire_scatter(c - 2, bs)

    drain_scatter(2)
    drain_scatter(3)

    plsc.subcore_barrier()
    pltpu.sync_copy(
        acc_sh.at[pl.ds(sid * ROWS_PT, ROWS_PT)],
        acc_out.at[cid, pl.ds(sid * ROWS_PT, ROWS_PT)],
    )


def _mm_body(acc_ref, x_ref, dinv_ref, w1_ref, b1_ref, w2_ref, u_ref):
    di = dinv_ref[...]
    a2 = acc_ref[...]
    a = (jnp.concatenate([a2[0], a2[1]], axis=1) + x_ref[...] * di) * di
    h = jnp.dot(a, w1_ref[...], preferred_element_type=jnp.float32)
    h = jnp.maximum(h + b1_ref[...], 0.0)
    t = jnp.dot(h, w2_ref[...], preferred_element_type=jnp.float32)
    u_ref[...] = t * di


def _mm(acc2, xp, dinv, w1r, b1r, w2):
    bm = 512
    return pl.pallas_call(
        _mm_body,
        grid=(NP // bm,),
        in_specs=[
            pl.BlockSpec((2, bm, DH), lambda i: (0, i, 0)),
            pl.BlockSpec((bm, DR), lambda i: (i, 0)),
            pl.BlockSpec((bm, 1), lambda i: (i, 0)),
            pl.BlockSpec((DR, HID), lambda i: (0, 0)),
            pl.BlockSpec((1, HID), lambda i: (0, 0)),
            pl.BlockSpec((HID, 1), lambda i: (0, 0)),
        ],
        out_specs=pl.BlockSpec((bm, 1), lambda i: (i, 0)),
        out_shape=jax.ShapeDtypeStruct((NP, 1), jnp.float32),
    )(acc2, xp, dinv, w1r, b1r, w2)


@functools.partial(
    pl.kernel,
    out_type=jax.ShapeDtypeStruct((NP,), jnp.float32),
    mesh=_mesh,
    compiler_params=_sc_params_nl,
    scratch_types=[
        pltpu.VMEM((CPT, K), jnp.int32),
        pltpu.VMEM((CPT, K), jnp.int32),
        pltpu.VMEM((CPT, K), jnp.float32),
        pltpu.VMEM((NP,), jnp.float32),
        pltpu.VMEM((ROWS_PT,), jnp.float32),
        pltpu.VMEM((ROWS_PW,), jnp.float32),
        pltpu.VMEM((ROWS_PW,), jnp.float32),
        pltpu.VMEM((ROWS_PW,), jnp.float32),
        pltpu.VMEM((ROWS_PW,), jnp.float32),
        pltpu.VMEM((16,), jnp.float32),
        pltpu.SemaphoreType.DMA,
        pltpu.VMEM_SHARED((NP,), jnp.float32),
    ],
)
def _s2_kernel(src_hbm, dst_hbm, u_hbm, dinv_hbm, b2_hbm, o_hbm,
               si_v, di_v, upd_v, u_v, z_v, s2b_v, ub_v, db_v, ob_v, b2_v,
               ssem, acc_sh):
    cid = lax.axis_index("c")
    sid = lax.axis_index("s")
    wid = cid * 16 + sid

    _zero_1d(z_v, ROWS_PT)
    pltpu.sync_copy(z_v, acc_sh.at[pl.ds(sid * ROWS_PT, ROWS_PT)])
    pltpu.sync_copy(src_hbm.at[sid], si_v)
    pltpu.sync_copy(dst_hbm.at[sid], di_v)
    pltpu.sync_copy(b2_hbm, b2_v)
    pltpu.sync_copy(u_hbm, u_v)

    @pl.loop(0, CPT)
    def _(c):
        @pl.loop(0, K // 16)
        def _(j):
            sidx = si_v[c, pl.ds(j * 16, 16)]
            upd_v[c, pl.ds(j * 16, 16)] = plsc.load_gather(u_v, [sidx])

    plsc.subcore_barrier()

    @pl.loop(0, CPT, step=8)
    def _(c0):
        descs = [
            pltpu.async_copy(upd_v.at[c0 + b], acc_sh.at[di_v.at[c0 + b]],
                             ssem, add=True)
            for b in range(8)
        ]
        for d in descs:
            d.wait()

    plsc.subcore_barrier()

    base = wid * ROWS_PW
    pltpu.sync_copy(acc_sh.at[pl.ds(base, ROWS_PW)], s2b_v)
    pltpu.sync_copy(u_hbm.at[pl.ds(base, ROWS_PW)], ub_v)
    pltpu.sync_copy(dinv_hbm.at[pl.ds(base, ROWS_PW)], db_v)

    @pl.loop(0, ROWS_PW // 16)
    def _(k):
        s2 = s2b_v[pl.ds(k * 16, 16)]
        uu = ub_v[pl.ds(k * 16, 16)]
        dd = db_v[pl.ds(k * 16, 16)]
        b2 = b2_v[pl.ds(0, 16)]
        zz = dd * (s2 + uu) + b2
        ob_v[pl.ds(k * 16, 16)] = 1.0 / (1.0 + jnp.exp(-zz))

    pltpu.sync_copy(ob_v, o_hbm.at[pl.ds(base, ROWS_PW)])


def kernel(x, edge_index, W1, b1, W2, b2):
    src = edge_index[0].astype(jnp.int32)
    dst = edge_index[1].astype(jnp.int32)
    pad_e = EP - E
    fdst = 10016 + jnp.arange(pad_e, dtype=jnp.int32) % 224
    fsrc = jnp.arange(pad_e, dtype=jnp.int32) % N
    src_p = jnp.concatenate([src, fsrc])
    dst_p = jnp.concatenate([dst, fdst])
    src_t = src_p.reshape(16, CPT, K)
    dst_t = dst_p.reshape(16, CPT, K)
    src_ks = src_p.reshape(16, CPS, KS)
    dst_ks = dst_p.reshape(16, CPS, KS)

    xp = jnp.pad(x, ((0, NP - N), (0, DR - DIN)))
    w1r = jnp.pad(W1, ((0, DR - DIN), (0, 0)))
    b1r = b1.reshape(1, HID)
    b2v = jnp.broadcast_to(b2, (16,))

    dinv = _deg_kernel(dst_t)
    y2 = _y2_kernel(xp, dinv)
    acc2 = _s1_kernel(src_ks, dst_ks, y2)
    u = _mm(acc2, xp, dinv.reshape(NP, 1), w1r, b1r, W2)
    o = _s2_kernel(src_t, dst_t, u.reshape(NP), dinv, b2v)
    return o[:N].reshape(N, 1)


# speedup vs baseline: 36.3455x; 1.0494x over previous
import functools

import jax
import jax.numpy as jnp
from jax import lax
from jax.experimental import pallas as pl
from jax.experimental.pallas import tpu as pltpu
from jax.experimental.pallas import tpu_sc as plsc

N = 10000
NP = 10240
DIN = 165
DR = 192
DH = 96
HID = 360
E = 320000
K = 128
CPW = 84
EP = 32 * CPW * K
CPT = EP // K // 16
KS = 64
CPS = EP // KS // 16
NW = 32
ROWS_PT = NP // 16
ROWS_PW = NP // NW

_mesh = plsc.VectorSubcoreMesh(
    core_axis_name="c", subcore_axis_name="s", num_cores=2, num_subcores=16
)
_sc_params = pltpu.CompilerParams(use_tc_tiling_on_sc=False)
_sc_params_nl = pltpu.CompilerParams(use_tc_tiling_on_sc=False,
                                     needs_layout_passes=False)


def _zero_1d(ref, n):
    @pl.loop(0, n // 16)
    def _(i):
        ref[pl.ds(i * 16, 16)] = jnp.zeros((16,), jnp.float32)


@functools.partial(
    pl.kernel,
    out_type=jax.ShapeDtypeStruct((NP,), jnp.float32),
    mesh=_mesh,
    compiler_params=_sc_params_nl,
    scratch_types=[
        pltpu.VMEM((CPT, K), jnp.int32),
        pltpu.VMEM((K,), jnp.float32),
        pltpu.VMEM((ROWS_PT,), jnp.float32),
        pltpu.VMEM((ROWS_PW,), jnp.float32),
        pltpu.SemaphoreType.DMA,
        pltpu.VMEM_SHARED((NP,), jnp.float32),
    ],
)
def _deg_kernel(dst_hbm, dinv_out, idx_v, ones_v, z_v, dinvb_v, hsem, acc_sh):
    cid = lax.axis_index("c")
    sid = lax.axis_index("s")
    wid = cid * 16 + sid

    @pl.loop(0, K // 16)
    def _(i):
        ones_v[pl.ds(i * 16, 16)] = jnp.ones((16,), jnp.float32)

    _zero_1d(z_v, ROWS_PT)
    pltpu.sync_copy(z_v, acc_sh.at[pl.ds(sid * ROWS_PT, ROWS_PT)])
    pltpu.sync_copy(dst_hbm.at[sid], idx_v)
    plsc.subcore_barrier()

    def fire(c):
        pltpu.async_copy(ones_v, acc_sh.at[idx_v.at[c]], hsem, add=True)

    def drain():
        pltpu.make_async_copy(ones_v, acc_sh.at[idx_v.at[0]], hsem).wait()

    fire(0)
    fire(1)

    @pl.loop(2, CPT)
    def _(c):
        drain()
        fire(c)

    drain()
    drain()
    plsc.subcore_barrier()

    pltpu.sync_copy(acc_sh.at[pl.ds(wid * ROWS_PW, ROWS_PW)], z_v.at[pl.ds(0, ROWS_PW)])

    @pl.loop(0, ROWS_PW // 16)
    def _(k):
        d = z_v[pl.ds(k * 16, 16)] + 1.0
        i = plsc.bitcast(d, jnp.int32)
        i = jnp.full((16,), 0x5F3759DF, jnp.int32) - lax.shift_right_logical(i, 1)
        r = plsc.bitcast(i, jnp.float32)
        r = r * (1.5 - 0.5 * d * r * r)
        r = r * (1.5 - 0.5 * d * r * r)
        r = r * (1.5 - 0.5 * d * r * r)
        dinvb_v[pl.ds(k * 16, 16)] = r

    pltpu.sync_copy(dinvb_v, dinv_out.at[pl.ds(wid * ROWS_PW, ROWS_PW)])


@functools.partial(
    pl.kernel,
    out_type=jax.ShapeDtypeStruct((2 * NP, DH), jnp.float32),
    mesh=_mesh,
    compiler_params=_sc_params_nl,
    scratch_types=[
        pltpu.VMEM((64, DR), jnp.float32),
        pltpu.VMEM((128, DH), jnp.float32),
        pltpu.VMEM((ROWS_PW,), jnp.float32),
        pltpu.SemaphoreType.DMA,
    ],
)
def _y2_kernel(xp_hbm, dinv_hbm, y2_out, xb_v, yb_v, dinvb_v, sem):
    cid = lax.axis_index("c")
    sid = lax.axis_index("s")
    wid = cid * 16 + sid
    base = wid * ROWS_PW
    pltpu.sync_copy(dinv_hbm.at[pl.ds(base, ROWS_PW)], dinvb_v)

    @pl.loop(0, ROWS_PW // 64)
    def _(j):
        pltpu.sync_copy(xp_hbm.at[pl.ds(base + j * 64, 64)], xb_v)

        @pl.loop(0, 64)
        def _(n):
            dv = plsc.load_gather(dinvb_v, [jnp.full((16,), j * 64 + n, jnp.int32)])
            for k in range(6):
                yb_v[2 * n, pl.ds(k * 16, 16)] = xb_v[n, pl.ds(k * 16, 16)] * dv
                yb_v[2 * n + 1, pl.ds(k * 16, 16)] = (
                    xb_v[n, pl.ds(DH + k * 16, 16)] * dv)

        pltpu.sync_copy(yb_v, y2_out.at[pl.ds(2 * base + j * 128, 128)])


@functools.partial(
    pl.kernel,
    out_type=jax.ShapeDtypeStruct((2, NP, DH), jnp.float32),
    mesh=_mesh,
    compiler_params=_sc_params,
    scratch_types=[
        pltpu.VMEM((CPS, KS), jnp.int32),
        pltpu.VMEM((CPS, KS), jnp.int32),
        [pltpu.VMEM((KS, DH), jnp.float32)] * 4,
        [pltpu.SemaphoreType.DMA] * 4,
        [pltpu.SemaphoreType.DMA] * 4,
        pltpu.VMEM_SHARED((NP, DH), jnp.float32),
    ],
)
def _s1_kernel(src_hbm, dst_hbm, y_hbm, acc_out, si_v, di_v,
               rows, gsem, ssem, acc_sh):
    cid = lax.axis_index("c")
    sid = lax.axis_index("s")

    @pl.loop(0, KS)
    def _(r):
        @pl.loop(0, DH // 16)
        def _(j):
            rows[0][r, pl.ds(j * 16, 16)] = jnp.zeros((16,), jnp.float32)

    @pl.loop(0, ROWS_PT // KS)
    def _(b):
        pltpu.sync_copy(rows[0], acc_sh.at[pl.ds(sid * ROWS_PT + b * KS, KS)])

    pltpu.sync_copy(src_hbm.at[sid], si_v)
    pltpu.sync_copy(dst_hbm.at[sid], di_v)

    @pl.loop(0, CPS)
    def _(r):
        @pl.loop(0, KS // 16)
        def _(j):
            v = si_v[r, pl.ds(j * 16, 16)]
            si_v[r, pl.ds(j * 16, 16)] = v * 2 + cid

    plsc.subcore_barrier()

    def fire_gather(c, b):
        pltpu.async_copy(y_hbm.at[si_v.at[c]], rows[b], gsem[b])

    def drain_gather(b):
        pltpu.make_async_copy(y_hbm.at[si_v.at[0]], rows[b], gsem[b]).wait()

    def fire_scatter(c, b):
        pltpu.async_copy(rows[b], acc_sh.at[di_v.at[c]], ssem[b], add=True)

    def drain_scatter(b):
        pltpu.make_async_copy(rows[b], acc_sh.at[di_v.at[0]], ssem[b]).wait()

    fire_gather(0, 0)
    fire_gather(1, 1)
    fire_gather(2, 2)

    @pl.loop(3, CPS + 3, step=4)
    def _(c0):
        for o in range(4):
            c = c0 + o
            bg = (3 + o) % 4
            bs = o % 4

            @pl.when(c >= 4)
            def _():
                drain_scatter(bg)

            @pl.when(c < CPS)
            def _():
                fire_gather(c, bg)

            drain_gather(bs)
            fire_scatter(c - 3, bs)

    drain_scatter(3)

    plsc.subcore_barrier()
    pltpu.sync_copy(
        acc_sh.at[pl.ds(sid * ROWS_PT, ROWS_PT)],
        acc_out.at[cid, pl.ds(sid * ROWS_PT, ROWS_PT)],
    )


def _mm_body(acc_ref, x_ref, dinv_ref, w1_ref, b1_ref, w2_ref, u_ref):
    di = dinv_ref[...]
    a2 = acc_ref[...]
    a = (jnp.concatenate([a2[0], a2[1]], axis=1) + x_ref[...] * di) * di
    h = jnp.dot(a, w1_ref[...], preferred_element_type=jnp.float32)
    h = jnp.maximum(h + b1_ref[...], 0.0)
    t = jnp.dot(h, w2_ref[...], preferred_element_type=jnp.float32)
    u_ref[...] = t * di


def _mm(acc2, xp, dinv, w1r, b1r, w2):
    bm = 512
    return pl.pallas_call(
        _mm_body,
        grid=(NP // bm,),
        in_specs=[
            pl.BlockSpec((2, bm, DH), lambda i: (0, i, 0)),
            pl.BlockSpec((bm, DR), lambda i: (i, 0)),
            pl.BlockSpec((bm, 1), lambda i: (i, 0)),
            pl.BlockSpec((DR, HID), lambda i: (0, 0)),
            pl.BlockSpec((1, HID), lambda i: (0, 0)),
            pl.BlockSpec((HID, 1), lambda i: (0, 0)),
        ],
        out_specs=pl.BlockSpec((bm, 1), lambda i: (i, 0)),
        out_shape=jax.ShapeDtypeStruct((NP, 1), jnp.float32),
    )(acc2, xp, dinv, w1r, b1r, w2)


@functools.partial(
    pl.kernel,
    out_type=jax.ShapeDtypeStruct((NP,), jnp.float32),
    mesh=_mesh,
    compiler_params=_sc_params_nl,
    scratch_types=[
        pltpu.VMEM((CPT, K), jnp.int32),
        pltpu.VMEM((CPT, K), jnp.int32),
        pltpu.VMEM((CPT, K), jnp.float32),
        pltpu.VMEM((NP,), jnp.float32),
        pltpu.VMEM((ROWS_PT,), jnp.float32),
        pltpu.VMEM((ROWS_PW,), jnp.float32),
        pltpu.VMEM((ROWS_PW,), jnp.float32),
        pltpu.VMEM((ROWS_PW,), jnp.float32),
        pltpu.VMEM((ROWS_PW,), jnp.float32),
        pltpu.VMEM((16,), jnp.float32),
        pltpu.SemaphoreType.DMA,
        pltpu.VMEM_SHARED((NP,), jnp.float32),
    ],
)
def _s2_kernel(src_hbm, dst_hbm, u_hbm, dinv_hbm, b2_hbm, o_hbm,
               si_v, di_v, upd_v, u_v, z_v, s2b_v, ub_v, db_v, ob_v, b2_v,
               ssem, acc_sh):
    cid = lax.axis_index("c")
    sid = lax.axis_index("s")
    wid = cid * 16 + sid

    _zero_1d(z_v, ROWS_PT)
    pltpu.sync_copy(z_v, acc_sh.at[pl.ds(sid * ROWS_PT, ROWS_PT)])
    pltpu.sync_copy(src_hbm.at[sid], si_v)
    pltpu.sync_copy(dst_hbm.at[sid], di_v)
    pltpu.sync_copy(b2_hbm, b2_v)
    pltpu.sync_copy(u_hbm, u_v)

    @pl.loop(0, CPT)
    def _(c):
        @pl.loop(0, K // 16)
        def _(j):
            sidx = si_v[c, pl.ds(j * 16, 16)]
            upd_v[c, pl.ds(j * 16, 16)] = plsc.load_gather(u_v, [sidx])

    plsc.subcore_barrier()

    @pl.loop(0, CPT, step=8)
    def _(c0):
        descs = [
            pltpu.async_copy(upd_v.at[c0 + b], acc_sh.at[di_v.at[c0 + b]],
                             ssem, add=True)
            for b in range(8)
        ]
        for d in descs:
            d.wait()

    plsc.subcore_barrier()

    base = wid * ROWS_PW
    pltpu.sync_copy(acc_sh.at[pl.ds(base, ROWS_PW)], s2b_v)
    pltpu.sync_copy(u_hbm.at[pl.ds(base, ROWS_PW)], ub_v)
    pltpu.sync_copy(dinv_hbm.at[pl.ds(base, ROWS_PW)], db_v)

    @pl.loop(0, ROWS_PW // 16)
    def _(k):
        s2 = s2b_v[pl.ds(k * 16, 16)]
        uu = ub_v[pl.ds(k * 16, 16)]
        dd = db_v[pl.ds(k * 16, 16)]
        b2 = b2_v[pl.ds(0, 16)]
        zz = dd * (s2 + uu) + b2
        ob_v[pl.ds(k * 16, 16)] = 1.0 / (1.0 + jnp.exp(-zz))

    pltpu.sync_copy(ob_v, o_hbm.at[pl.ds(base, ROWS_PW)])


def kernel(x, edge_index, W1, b1, W2, b2):
    src = edge_index[0].astype(jnp.int32)
    dst = edge_index[1].astype(jnp.int32)
    pad_e = EP - E
    fdst = 10016 + jnp.arange(pad_e, dtype=jnp.int32) % 224
    fsrc = jnp.arange(pad_e, dtype=jnp.int32) % N
    src_p = jnp.concatenate([src, fsrc])
    dst_p = jnp.concatenate([dst, fdst])
    src_t = src_p.reshape(16, CPT, K)
    dst_t = dst_p.reshape(16, CPT, K)
    src_ks = src_p.reshape(16, CPS, KS)
    dst_ks = dst_p.reshape(16, CPS, KS)

    xp = jnp.pad(x, ((0, NP - N), (0, DR - DIN)))
    w1r = jnp.pad(W1, ((0, DR - DIN), (0, 0)))
    b1r = b1.reshape(1, HID)
    b2v = jnp.broadcast_to(b2, (16,))

    dinv = _deg_kernel(dst_t)
    y2 = _y2_kernel(xp, dinv)
    acc2 = _s1_kernel(src_ks, dst_ks, y2)
    u = _mm(acc2, xp, dinv.reshape(NP, 1), w1r, b1r, W2)
    o = _s2_kernel(src_t, dst_t, u.reshape(NP), dinv, b2v)
    return o[:N].reshape(N, 1)
